# baseline (speedup 1.0000x reference)
import functools

import jax
import jax.numpy as jnp
from jax import lax
from jax.experimental import pallas as pl
from jax.experimental.pallas import tpu as pltpu
from jax.experimental.pallas import tpu_sc as plsc

_NUM_TABLES = 4
_EMB = 32
_NW = 32
_CHUNK = 128
_LANES = 16


@functools.partial(jax.jit, static_argnames=("batch",))
def _gather_concat(idx2d, t0, t1, t2, t3, *, batch):
    b_per_w = batch // _NW
    n_chunks = b_per_w // _CHUNK

    mesh = plsc.VectorSubcoreMesh(core_axis_name="c", subcore_axis_name="s")

    @functools.partial(
        pl.kernel,
        out_type=jax.ShapeDtypeStruct((batch * _NUM_TABLES, _EMB),
                                      jnp.float32),
        mesh=mesh,
        scratch_types=[
            pltpu.VMEM((n_chunks, _CHUNK), jnp.int32),
            pltpu.VMEM((_NUM_TABLES * n_chunks, _CHUNK), jnp.int32),
            pltpu.VMEM((b_per_w, _EMB), jnp.float32),
            pltpu.VMEM((b_per_w, _EMB), jnp.float32),
            pltpu.VMEM((b_per_w, _EMB), jnp.float32),
            pltpu.VMEM((b_per_w, _EMB), jnp.float32),
            pltpu.SemaphoreType.DMA,
        ],
        compiler_params=pltpu.CompilerParams(use_tc_tiling_on_sc=False),
    )
    def k(idx_hbm, t0_hbm, t1_hbm, t2_hbm, t3_hbm, out_hbm,
          idx_v, sidx_v, r0, r1, r2, r3, sem):
        wid = lax.axis_index("s") * 2 + lax.axis_index("c")
        base = wid * b_per_w
        pltpu.sync_copy(idx_hbm.at[pl.ds(wid * n_chunks, n_chunks)], idx_v)
        gathers = []
        for t, r in ((t0_hbm, r0), (t1_hbm, r1), (t2_hbm, r2), (t3_hbm, r3)):
            for j in range(n_chunks):
                gathers.append(
                    pltpu.async_copy(
                        t.at[idx_v.at[j]],
                        r.at[pl.ds(j * _CHUNK, _CHUNK)],
                        sem,
                    ))
        lanes = lax.broadcasted_iota(jnp.int32, (_LANES,), 0)
        for j in range(n_chunks):
            for c in range(_CHUNK // _LANES):
                gv4 = (base + j * _CHUNK + c * _LANES) * _NUM_TABLES \
                    + lanes * _NUM_TABLES
                for m in range(_NUM_TABLES):
                    sidx_v[m * n_chunks + j, pl.ds(c * _LANES, _LANES)] = \
                        gv4 + m
        for g in gathers:
            g.wait()
        scatters = []
        for m, r in enumerate((r0, r1, r2, r3)):
            for j in range(n_chunks):
                scatters.append(
                    pltpu.async_copy(
                        r.at[pl.ds(j * _CHUNK, _CHUNK)],
                        out_hbm.at[sidx_v.at[m * n_chunks + j]],
                        sem,
                    ))
        for s in scatters:
            s.wait()

    return k(idx2d, t0, t1, t2, t3)


def kernel(indexes, table0, table1, table2, table3):
    batch = indexes.shape[0]
    idx2d = indexes.astype(jnp.int32).reshape(batch // _CHUNK, _CHUNK)
    out = _gather_concat(idx2d, table0, table1, table2, table3, batch=batch)
    return out.reshape(batch, _NUM_TABLES * _EMB)
